# baseline (speedup 1.0000x reference)
import functools

import jax
import jax.numpy as jnp
from jax.experimental import pallas as pl
from jax.experimental.pallas import tpu as pltpu

_HID = 1024
_SP = 8192
_K = 64
_NTOK = 2048


def _enc_body(x_ref, g_ref, b_ref, wenc_ref, benc_ref, a_ref):
    x = x_ref[...]
    mu = jnp.mean(x, axis=-1, keepdims=True)
    var = jnp.mean((x - mu) ** 2, axis=-1, keepdims=True)
    h = (x - mu) * jax.lax.rsqrt(var + 1e-5) * g_ref[...] + b_ref[...]
    pre = jax.lax.dot_general(
        h, wenc_ref[...], (((1,), (1,)), ((), ())),
        preferred_element_type=jnp.float32,
    ) + benc_ref[...]
    a_ref[...] = jnp.maximum(pre, 0.0)


def _encode(x, ln_gamma, ln_beta, W_enc, b_enc):
    TM = 256
    TN = 2048
    grid = (_NTOK // TM, _SP // TN)
    return pl.pallas_call(
        _enc_body,
        grid=grid,
        in_specs=[
            pl.BlockSpec((TM, _HID), lambda i, j: (i, 0)),
            pl.BlockSpec((1, _HID), lambda i, j: (0, 0)),
            pl.BlockSpec((1, _HID), lambda i, j: (0, 0)),
            pl.BlockSpec((TN, _HID), lambda i, j: (j, 0)),
            pl.BlockSpec((1, TN), lambda i, j: (0, j)),
        ],
        out_specs=pl.BlockSpec((TM, TN), lambda i, j: (i, j)),
        out_shape=jax.ShapeDtypeStruct((_NTOK, _SP), jnp.float32),
    )(x, ln_gamma.reshape(1, _HID), ln_beta.reshape(1, _HID),
      W_enc, b_enc.reshape(1, _SP))


def kernel(x, ln_gamma, ln_beta, W_enc, b_enc, W_dec):
    activated = _encode(x, ln_gamma, ln_beta, W_enc, b_enc)
    topk_vals, topk_idx = jax.lax.top_k(activated, _K)
    rows = jnp.arange(activated.shape[0])[:, None]
    sparse_codes = jnp.zeros_like(activated).at[rows, topk_idx].set(topk_vals)
    reconstruction = sparse_codes @ W_dec.T
    return reconstruction, sparse_codes, topk_idx
